# Initial kernel scaffold; baseline (speedup 1.0000x reference)
import jax
import jax.numpy as jnp
from jax.experimental import pallas as pl


def kernel(input, bank):
    raise NotImplementedError("write your pallas kernel here")


# speedup vs baseline: 3.2383x; 3.2383x over previous
import jax
import jax.numpy as jnp
from jax.experimental import pallas as pl
from jax.experimental.pallas import tpu as pltpu

_FEA_DIM = 256
_BANK_DIM = 1024
_SHRINK = 0.0025
_BLOCK_M = 1024


def _fused_body(x_ref, bank_ref, o_ref):
    x = x_ref[...]
    bank = bank_ref[...]
    a = jax.lax.dot_general(
        x, bank, (((1,), (1,)), ((), ())), preferred_element_type=jnp.float32
    )
    m = jnp.max(a, axis=1, keepdims=True)
    e = jnp.exp(a - m)
    p = e * (1.0 / jnp.sum(e, axis=1, keepdims=True))
    s = jnp.maximum(p - _SHRINK, 0.0)
    e2 = jnp.exp(s)
    w = e2 * (1.0 / jnp.sum(e2, axis=1, keepdims=True))
    o = jnp.dot(w, bank, preferred_element_type=jnp.float32)
    o_ref[...] = jnp.tanh(o)


def kernel(input, bank):
    n, f = input.shape
    grid = (n // _BLOCK_M,)
    return pl.pallas_call(
        _fused_body,
        grid=grid,
        in_specs=[
            pl.BlockSpec((_BLOCK_M, f), lambda i: (i, 0)),
            pl.BlockSpec((_BANK_DIM, f), lambda i: (0, 0)),
        ],
        out_specs=pl.BlockSpec((_BLOCK_M, f), lambda i: (i, 0)),
        out_shape=jax.ShapeDtypeStruct((n, f), jnp.float32),
        compiler_params=pltpu.CompilerParams(
            dimension_semantics=("arbitrary",),
        ),
    )(input, bank)


# speedup vs baseline: 3.2636x; 1.0078x over previous
import jax
import jax.numpy as jnp
from jax.experimental import pallas as pl
from jax.experimental.pallas import tpu as pltpu

_FEA_DIM = 256
_BANK_DIM = 1024
_SHRINK = 0.0025
_BLOCK_M = 1024


def _fused_body(x_ref, bank_ref, o_ref):
    x = x_ref[...].astype(jnp.bfloat16)
    bank = bank_ref[...].astype(jnp.bfloat16)
    a = jax.lax.dot_general(
        x, bank, (((1,), (1,)), ((), ())), preferred_element_type=jnp.float32
    )
    m = jnp.max(a, axis=1, keepdims=True)
    e = jnp.exp(a - m)
    p = e * (1.0 / jnp.sum(e, axis=1, keepdims=True))
    s = jnp.maximum(p - _SHRINK, 0.0)
    e2 = jnp.exp(s)
    w = e2 * (1.0 / jnp.sum(e2, axis=1, keepdims=True))
    o = jnp.dot(w.astype(jnp.bfloat16), bank, preferred_element_type=jnp.float32)
    o_ref[...] = jnp.tanh(o)


def kernel(input, bank):
    n, f = input.shape
    grid = (n // _BLOCK_M,)
    return pl.pallas_call(
        _fused_body,
        grid=grid,
        in_specs=[
            pl.BlockSpec((_BLOCK_M, f), lambda i: (i, 0)),
            pl.BlockSpec((_BANK_DIM, f), lambda i: (0, 0)),
        ],
        out_specs=pl.BlockSpec((_BLOCK_M, f), lambda i: (i, 0)),
        out_shape=jax.ShapeDtypeStruct((n, f), jnp.float32),
        compiler_params=pltpu.CompilerParams(
            dimension_semantics=("arbitrary",),
        ),
    )(input, bank)


# speedup vs baseline: 3.4290x; 1.0507x over previous
import jax
import jax.numpy as jnp
from jax.experimental import pallas as pl
from jax.experimental.pallas import tpu as pltpu

_FEA_DIM = 256
_BANK_DIM = 1024
_SHRINK = 0.0025
_BLOCK_M = 1024


def _fused_body(x_ref, bank_ref, o_ref):
    x = x_ref[...].astype(jnp.bfloat16)
    bank = bank_ref[...].astype(jnp.bfloat16)
    a = jax.lax.dot_general(
        x, bank, (((1,), (1,)), ((), ())), preferred_element_type=jnp.float32
    )
    m = jnp.max(a, axis=1, keepdims=True)
    e = jnp.exp(a - m)
    p = e * (1.0 / jnp.sum(e, axis=1, keepdims=True))
    s = jnp.maximum(p - _SHRINK, 0.0)
    e2 = jnp.exp(s)
    inv_z2 = 1.0 / jnp.sum(e2, axis=1, keepdims=True)
    o = jnp.dot(e2.astype(jnp.bfloat16), bank, preferred_element_type=jnp.float32)
    o_ref[...] = jnp.tanh(o * inv_z2)


def kernel(input, bank):
    n, f = input.shape
    grid = (n // _BLOCK_M,)
    return pl.pallas_call(
        _fused_body,
        grid=grid,
        in_specs=[
            pl.BlockSpec((_BLOCK_M, f), lambda i: (i, 0)),
            pl.BlockSpec((_BANK_DIM, f), lambda i: (0, 0)),
        ],
        out_specs=pl.BlockSpec((_BLOCK_M, f), lambda i: (i, 0)),
        out_shape=jax.ShapeDtypeStruct((n, f), jnp.float32),
        compiler_params=pltpu.CompilerParams(
            dimension_semantics=("parallel",),
        ),
    )(input, bank)
